# Initial kernel scaffold; baseline (speedup 1.0000x reference)
import jax
import jax.numpy as jnp
from jax.experimental import pallas as pl


def kernel(x, conv1_w, conv1_b, gcn_w, gcn_b, conv2_w, conv2_b, ln_gamma, ln_beta, filter_vals, filter_rows, filter_cols):
    raise NotImplementedError("write your pallas kernel here")


# speedup vs baseline: 1.5166x; 1.5166x over previous
import functools

import jax
import jax.numpy as jnp
from jax import lax
from jax.experimental import pallas as pl
from jax.experimental.pallas import tpu as pltpu
from jax.experimental.pallas import tpu_sc as plsc

B, C, T, NV, KT, NNZ = 4, 32, 12, 10000, 3, 160000
TOUT = T - KT + 1
TOUT2 = TOUT - KT + 1


VB = 1280
NVB = -(-NV // VB)


def _k1_body(x, w, b, o):
    for t in range(TOUT):
        acc = b[...]
        for k in range(KT):
            acc = acc + jnp.dot(w[:, :, k], x[0, :, t + k, :],
                                preferred_element_type=jnp.float32)
        p = acc[:C, :]
        q = acc[C:, :]
        o[0, :, t, :] = (p + x[0, :, t + KT - 1, :]) * jax.nn.sigmoid(q)


def _conv1_glu(x, w1, b1):
    return pl.pallas_call(
        _k1_body,
        grid=(B, NVB),
        in_specs=[
            pl.BlockSpec((1, C, T, VB), lambda b, v: (b, 0, 0, v)),
            pl.BlockSpec((2 * C, C, KT), lambda b, v: (0, 0, 0)),
            pl.BlockSpec((2 * C, 1), lambda b, v: (0, 0)),
        ],
        out_specs=pl.BlockSpec((1, C, TOUT, VB), lambda b, v: (b, 0, 0, v)),
        out_shape=jax.ShapeDtypeStruct((B, C, TOUT, NV), jnp.float32),
    )(x, w1, b1)


CT = 128
NCT = 1280 // CT
TILES_PER_SC = NCT // 2
NS = 16
EK = 80
EPT = NNZ // NS
NCHUNK = EPT // EK
RPAD = 10240
RPT = RPAD // NS
ZR = 128


def _spmm_body(f3, rows_h, cols_h, vals_h, out_h,
               cols_v, rows_v, vals_v, buf, zbuf, acc):
    cid = lax.axis_index("c")
    sid = lax.axis_index("s")

    def _z(i, _):
        for r in range(CT // 16):
            zbuf[i, pl.ds(r * 16, 16)] = jnp.zeros((16,), jnp.float32)
        return 0
    lax.fori_loop(0, ZR, _z, 0)

    ebase = pl.multiple_of(sid * EPT, 8)

    for j in range(TILES_PER_SC):
        ct = cid * TILES_PER_SC + j
        cbase = pl.multiple_of(ct * CT, CT)

        for z in range(RPT // ZR):
            pltpu.sync_copy(zbuf, acc.at[pl.ds(sid * RPT + z * ZR, ZR)])
        plsc.subcore_barrier()

        def _chunk(i, _):
            base = pl.multiple_of(ebase + i * EK, 8)
            pltpu.sync_copy(cols_h.at[pl.ds(base, EK)], cols_v)
            pltpu.sync_copy(rows_h.at[pl.ds(base, EK)], rows_v)
            pltpu.sync_copy(vals_h.at[pl.ds(base, EK)], vals_v)
            pltpu.sync_copy(f3.at[cols_v, pl.ds(cbase, CT)], buf)

            def _scale(e, _):
                g = (e // 16) * 16
                v16 = vals_v[pl.ds(g, 16)]
                lane = lax.broadcast(e - g, (16,))
                splat = lax.gather(
                    v16, lane[:, None],
                    lax.GatherDimensionNumbers(offset_dims=(),
                                               collapsed_slice_dims=(0,),
                                               start_index_map=(0,)),
                    (1,), mode=lax.GatherScatterMode.PROMISE_IN_BOUNDS)
                for r in range(CT // 16):
                    buf[e, pl.ds(r * 16, 16)] = buf[e, pl.ds(r * 16, 16)] * splat
                return 0
            lax.fori_loop(0, EK, _scale, 0)

            pltpu.sync_copy(buf, acc.at[rows_v], add=True)
            return 0
        lax.fori_loop(0, NCHUNK, _chunk, 0)
        plsc.subcore_barrier()

        @pl.when(sid < NS - 1)
        def _flush_main():
            pltpu.sync_copy(acc.at[pl.ds(sid * RPT, RPT)],
                            out_h.at[pl.ds(sid * RPT, RPT), pl.ds(cbase, CT)])

        @pl.when(sid == NS - 1)
        def _flush_last():
            nlast = NV - (NS - 1) * RPT
            pltpu.sync_copy(acc.at[pl.ds((NS - 1) * RPT, nlast)],
                            out_h.at[pl.ds((NS - 1) * RPT, nlast), pl.ds(cbase, CT)])
        plsc.subcore_barrier()


def _spmm_sc(f3, rows, cols, vals):
    fn = pl.kernel(
        _spmm_body,
        mesh=plsc.VectorSubcoreMesh(core_axis_name="c", subcore_axis_name="s"),
        out_type=jax.ShapeDtypeStruct((NV, NCT * CT), jnp.float32),
        scratch_types=[
            pltpu.VMEM((EK,), jnp.int32),
            pltpu.VMEM((EK,), jnp.int32),
            pltpu.VMEM((EK,), jnp.float32),
            pltpu.VMEM((EK, CT), jnp.float32),
            pltpu.VMEM((ZR, CT), jnp.float32),
            pltpu.VMEM_SHARED((RPAD, CT), jnp.float32),
        ],
    )
    return fn(f3, rows, cols, vals)


def _ka_body(hq, x1, gw, gb, o):
    for t in range(TOUT):
        g = lax.dot_general(gw[...], hq[0, t, :, :],
                            (((0,), (1,)), ((), ())),
                            preferred_element_type=jnp.float32)
        o[0, t, :, :] = jax.nn.relu(g + gb[...] + x1[0, :, t, :])


def _gcn_residual_relu(hq, x1, gw, gb):
    return pl.pallas_call(
        _ka_body,
        grid=(B, NVB),
        in_specs=[
            pl.BlockSpec((1, TOUT, VB, C), lambda b, v: (b, 0, v, 0)),
            pl.BlockSpec((1, C, TOUT, VB), lambda b, v: (b, 0, 0, v)),
            pl.BlockSpec((C, C), lambda b, v: (0, 0)),
            pl.BlockSpec((C, 1), lambda b, v: (0, 0)),
        ],
        out_specs=pl.BlockSpec((1, TOUT, C, VB), lambda b, v: (b, 0, 0, v)),
        out_shape=jax.ShapeDtypeStruct((B, TOUT, C, NV), jnp.float32),
    )(hq, x1, gw, gb)


def _kb_body(xr, w2, b2, gT, bT, o):
    for t in range(TOUT2):
        acc = b2[...]
        for k in range(KT):
            acc = acc + jnp.dot(w2[:, :, k], xr[0, t + k, :, :],
                                preferred_element_type=jnp.float32)
        x2 = jax.nn.relu(acc + xr[0, t + KT - 1, :, :])
        mean = jnp.mean(x2)
        var = jnp.mean((x2 - mean) ** 2)
        o[0, t, :, :] = (x2 - mean) * jax.lax.rsqrt(var + 1e-5) * gT[...] + bT[...]


def _conv2_ln(xr, w2, b2, gT, bT):
    return pl.pallas_call(
        _kb_body,
        grid=(B,),
        in_specs=[
            pl.BlockSpec((1, TOUT, C, NV), lambda b: (b, 0, 0, 0)),
            pl.BlockSpec((C, C, KT), lambda b: (0, 0, 0)),
            pl.BlockSpec((C, 1), lambda b: (0, 0)),
            pl.BlockSpec((C, NV), lambda b: (0, 0)),
            pl.BlockSpec((C, NV), lambda b: (0, 0)),
        ],
        out_specs=pl.BlockSpec((1, TOUT2, C, NV), lambda b: (b, 0, 0, 0)),
        out_shape=jax.ShapeDtypeStruct((B, TOUT2, C, NV), jnp.float32),
    )(xr, w2, b2, gT, bT)


def kernel(x, conv1_w, conv1_b, gcn_w, gcn_b, conv2_w, conv2_b,
           ln_gamma, ln_beta, filter_vals, filter_rows, filter_cols):
    x1 = _conv1_glu(x, conv1_w[:, :, :, 0], conv1_b.reshape(2 * C, 1))
    f3 = x1.reshape(NV, NCT * CT)
    h3 = _spmm_sc(f3, filter_rows, filter_cols, filter_vals)
    hq = h3.reshape(B, TOUT, NV, C)
    xr = _gcn_residual_relu(hq, x1, gcn_w, gcn_b.reshape(C, 1))
    out = _conv2_ln(xr, conv2_w[:, :, :, 0], conv2_b.reshape(C, 1),
                    ln_gamma.T, ln_beta.T)
    return out.transpose(0, 2, 1, 3)


# speedup vs baseline: 2.4843x; 1.6381x over previous
import functools

import jax
import jax.numpy as jnp
from jax import lax
from jax.experimental import pallas as pl
from jax.experimental.pallas import tpu as pltpu
from jax.experimental.pallas import tpu_sc as plsc

B, C, T, NV, KT, NNZ = 4, 32, 12, 10000, 3, 160000
TOUT = T - KT + 1
TOUT2 = TOUT - KT + 1


VB = 1280
NVB = -(-NV // VB)


def _k1_body(x, w, b, o):
    for t in range(TOUT):
        acc = b[...]
        for k in range(KT):
            acc = acc + jnp.dot(w[:, :, k], x[0, :, t + k, :],
                                preferred_element_type=jnp.float32)
        p = acc[:C, :]
        q = acc[C:, :]
        o[0, :, t, :] = (p + x[0, :, t + KT - 1, :]) * jax.nn.sigmoid(q)


def _conv1_glu(x, w1, b1):
    return pl.pallas_call(
        _k1_body,
        grid=(B, NVB),
        in_specs=[
            pl.BlockSpec((1, C, T, VB), lambda b, v: (b, 0, 0, v)),
            pl.BlockSpec((2 * C, C, KT), lambda b, v: (0, 0, 0)),
            pl.BlockSpec((2 * C, 1), lambda b, v: (0, 0)),
        ],
        out_specs=pl.BlockSpec((1, C, TOUT, VB), lambda b, v: (b, 0, 0, v)),
        out_shape=jax.ShapeDtypeStruct((B, C, TOUT, NV), jnp.float32),
    )(x, w1, b1)


CT = 128
NCT = 1280 // CT
TILES_PER_SC = NCT // 2
NS = 16
EK = 80
EPT = NNZ // NS
NBLK = 5
CBLK = 25
RPAD = 10240
RPT = RPAD // NS


def _spmm_body(f3, rows_h, cols_h, vals_h, out_h,
               cols_v, rows_v, vals_v, bufa, bufb, acc, sema, semb):
    cid = lax.axis_index("c")
    sid = lax.axis_index("s")

    def _zero_bufa(i, _):
        for r in range(CT // 16):
            bufa[i, pl.ds(r * 16, 16)] = jnp.zeros((16,), jnp.float32)
        return 0

    for j in range(TILES_PER_SC):
        ct = cid * TILES_PER_SC + j
        cbase = pl.multiple_of(ct * CT, CT)

        def _gather_start(ref, i, buf, sem):
            pltpu.async_copy(f3.at[ref.at[i], pl.ds(cbase, CT)], buf, sem)

        def _gather_wait(ref, i, buf, sem):
            pltpu.make_async_copy(f3.at[ref.at[i], pl.ds(cbase, CT)],
                                  buf, sem).wait()

        def _scale_scatter(i, buf):
            def _scale(e, _):
                g = (e // 16) * 16
                v16 = vals_v[i, pl.ds(g, 16)]
                lane = lax.broadcast(e - g, (16,))
                splat = lax.gather(
                    v16, lane[:, None],
                    lax.GatherDimensionNumbers(offset_dims=(),
                                               collapsed_slice_dims=(0,),
                                               start_index_map=(0,)),
                    (1,), mode=lax.GatherScatterMode.PROMISE_IN_BOUNDS)
                for r in range(CT // 16):
                    buf[e, pl.ds(r * 16, 16)] = buf[e, pl.ds(r * 16, 16)] * splat
                return 0
            lax.fori_loop(0, EK, _scale, 0)
            pltpu.sync_copy(buf, acc.at[rows_v.at[i]], add=True)

        lax.fori_loop(0, EK, _zero_bufa, 0)
        for z in range(RPT // EK):
            pltpu.sync_copy(bufa, acc.at[pl.ds(sid * RPT + z * EK, EK)])
        plsc.subcore_barrier()

        def _blk(blk, _):
            pltpu.sync_copy(cols_h.at[sid, blk], cols_v)
            pltpu.sync_copy(rows_h.at[sid, blk], rows_v)
            pltpu.sync_copy(vals_h.at[sid, blk], vals_v)
            _gather_start(cols_v, 0, bufa, sema)

            def _pair(ii, _):
                i = ii * 2
                _gather_wait(cols_v, i, bufa, sema)
                _gather_start(cols_v, i + 1, bufb, semb)
                _scale_scatter(i, bufa)
                _gather_wait(cols_v, i + 1, bufb, semb)
                _gather_start(cols_v, i + 2, bufa, sema)
                _scale_scatter(i + 1, bufb)
                return 0
            lax.fori_loop(0, (CBLK - 1) // 2, _pair, 0)
            _gather_wait(cols_v, CBLK - 1, bufa, sema)
            _scale_scatter(CBLK - 1, bufa)
            return 0
        lax.fori_loop(0, NBLK, _blk, 0)
        plsc.subcore_barrier()

        @pl.when(sid < NS - 1)
        def _flush_main():
            pltpu.sync_copy(acc.at[pl.ds(sid * RPT, RPT)],
                            out_h.at[pl.ds(sid * RPT, RPT), pl.ds(cbase, CT)])

        @pl.when(sid == NS - 1)
        def _flush_last():
            nlast = NV - (NS - 1) * RPT
            pltpu.sync_copy(acc.at[pl.ds((NS - 1) * RPT, nlast)],
                            out_h.at[pl.ds((NS - 1) * RPT, nlast), pl.ds(cbase, CT)])
        plsc.subcore_barrier()


def _spmm_sc(f3, rows, cols, vals):
    fn = pl.kernel(
        _spmm_body,
        mesh=plsc.VectorSubcoreMesh(core_axis_name="c", subcore_axis_name="s"),
        out_type=jax.ShapeDtypeStruct((NV, NCT * CT), jnp.float32),
        scratch_types=[
            pltpu.VMEM((CBLK, EK), jnp.int32),
            pltpu.VMEM((CBLK, EK), jnp.int32),
            pltpu.VMEM((CBLK, EK), jnp.float32),
            pltpu.VMEM((EK, CT), jnp.float32),
            pltpu.VMEM((EK, CT), jnp.float32),
            pltpu.VMEM_SHARED((RPAD, CT), jnp.float32),
            pltpu.SemaphoreType.DMA,
            pltpu.SemaphoreType.DMA,
        ],
    )
    return fn(f3, rows.reshape(NS, NBLK, CBLK, EK),
              cols.reshape(NS, NBLK, CBLK, EK),
              vals.reshape(NS, NBLK, CBLK, EK))


def _ka_body(hq, x1, gw, gb, o):
    for t in range(TOUT):
        g = lax.dot_general(gw[...], hq[0, t, :, :],
                            (((0,), (1,)), ((), ())),
                            preferred_element_type=jnp.float32)
        o[0, t, :, :] = jax.nn.relu(g + gb[...] + x1[0, :, t, :])


def _gcn_residual_relu(hq, x1, gw, gb):
    return pl.pallas_call(
        _ka_body,
        grid=(B, NVB),
        in_specs=[
            pl.BlockSpec((1, TOUT, VB, C), lambda b, v: (b, 0, v, 0)),
            pl.BlockSpec((1, C, TOUT, VB), lambda b, v: (b, 0, 0, v)),
            pl.BlockSpec((C, C), lambda b, v: (0, 0)),
            pl.BlockSpec((C, 1), lambda b, v: (0, 0)),
        ],
        out_specs=pl.BlockSpec((1, TOUT, C, VB), lambda b, v: (b, 0, 0, v)),
        out_shape=jax.ShapeDtypeStruct((B, TOUT, C, NV), jnp.float32),
    )(hq, x1, gw, gb)


def _kb_body(xr, w2, b2, gT, bT, o):
    for t in range(TOUT2):
        acc = b2[...]
        for k in range(KT):
            acc = acc + jnp.dot(w2[:, :, k], xr[0, t + k, :, :],
                                preferred_element_type=jnp.float32)
        x2 = jax.nn.relu(acc + xr[0, t + KT - 1, :, :])
        mean = jnp.mean(x2)
        var = jnp.mean((x2 - mean) ** 2)
        o[0, t, :, :] = (x2 - mean) * jax.lax.rsqrt(var + 1e-5) * gT[...] + bT[...]


def _conv2_ln(xr, w2, b2, gT, bT):
    return pl.pallas_call(
        _kb_body,
        grid=(B,),
        in_specs=[
            pl.BlockSpec((1, TOUT, C, NV), lambda b: (b, 0, 0, 0)),
            pl.BlockSpec((C, C, KT), lambda b: (0, 0, 0)),
            pl.BlockSpec((C, 1), lambda b: (0, 0)),
            pl.BlockSpec((C, NV), lambda b: (0, 0)),
            pl.BlockSpec((C, NV), lambda b: (0, 0)),
        ],
        out_specs=pl.BlockSpec((1, TOUT2, C, NV), lambda b: (b, 0, 0, 0)),
        out_shape=jax.ShapeDtypeStruct((B, TOUT2, C, NV), jnp.float32),
    )(xr, w2, b2, gT, bT)


def kernel(x, conv1_w, conv1_b, gcn_w, gcn_b, conv2_w, conv2_b,
           ln_gamma, ln_beta, filter_vals, filter_rows, filter_cols):
    x1 = _conv1_glu(x, conv1_w[:, :, :, 0], conv1_b.reshape(2 * C, 1))
    f3 = x1.reshape(NV, NCT * CT)
    h3 = _spmm_sc(f3, filter_rows, filter_cols, filter_vals)
    hq = h3.reshape(B, TOUT, NV, C)
    xr = _gcn_residual_relu(hq, x1, gcn_w, gcn_b.reshape(C, 1))
    out = _conv2_ln(xr, conv2_w[:, :, :, 0], conv2_b.reshape(C, 1),
                    ln_gamma.T, ln_beta.T)
    return out.transpose(0, 2, 1, 3)


# speedup vs baseline: 2.8756x; 1.1575x over previous
import functools

import jax
import jax.numpy as jnp
from jax import lax
from jax.experimental import pallas as pl
from jax.experimental.pallas import tpu as pltpu
from jax.experimental.pallas import tpu_sc as plsc

B, C, T, NV, KT, NNZ = 4, 32, 12, 10000, 3, 160000
TOUT = T - KT + 1
TOUT2 = TOUT - KT + 1


VB = 1280
NVB = -(-NV // VB)


def _k1_body(x, w, b, o):
    for t in range(TOUT):
        acc = b[...]
        for k in range(KT):
            acc = acc + jnp.dot(w[:, :, k], x[0, :, t + k, :],
                                preferred_element_type=jnp.float32)
        p = acc[:C, :]
        q = acc[C:, :]
        o[0, :, t, :] = (p + x[0, :, t + KT - 1, :]) * jax.nn.sigmoid(q)


def _conv1_glu(x, w1, b1):
    return pl.pallas_call(
        _k1_body,
        grid=(B, NVB),
        in_specs=[
            pl.BlockSpec((1, C, T, VB), lambda b, v: (b, 0, 0, v)),
            pl.BlockSpec((2 * C, C, KT), lambda b, v: (0, 0, 0)),
            pl.BlockSpec((2 * C, 1), lambda b, v: (0, 0)),
        ],
        out_specs=pl.BlockSpec((1, C, TOUT, VB), lambda b, v: (b, 0, 0, v)),
        out_shape=jax.ShapeDtypeStruct((B, C, TOUT, NV), jnp.float32),
    )(x, w1, b1)


CT = 128
NCT = 1280 // CT
TILES_PER_SC = NCT // 2
NS = 16
EK = 80
EPT = NNZ // NS
NBLK = 5
CBLK = 25
RPAD = 10240
RPT = RPAD // NS


def _spmm_body(f3, rows_h, cols_h, vals_h, out_h,
               cols_v, rows_v, vals_v, bufa, bufb, acc, sema, semb):
    cid = lax.axis_index("c")
    sid = lax.axis_index("s")

    def _zero_bufa(i, _):
        for r in range(CT // 16):
            bufa[i, pl.ds(r * 16, 16)] = jnp.zeros((16,), jnp.float32)
        return 0

    def _tile(j, _unused):
        ct = cid * TILES_PER_SC + j
        cbase = pl.multiple_of(ct * CT, CT)

        def _gather_start(ref, i, buf, sem):
            pltpu.async_copy(f3.at[ref.at[i], pl.ds(cbase, CT)], buf, sem)

        def _gather_wait(ref, i, buf, sem):
            pltpu.make_async_copy(f3.at[ref.at[i], pl.ds(cbase, CT)],
                                  buf, sem).wait()

        dnums = lax.GatherDimensionNumbers(offset_dims=(),
                                           collapsed_slice_dims=(0,),
                                           start_index_map=(0,))

        def _scale_scatter(i, buf):
            def _scale_group(g, _):
                e0 = g * 16
                v16 = vals_v[i, pl.ds(e0, 16)]
                for el in range(16):
                    splat = lax.gather(
                        v16, jnp.full((16, 1), el, jnp.int32), dnums, (1,),
                        mode=lax.GatherScatterMode.PROMISE_IN_BOUNDS)
                    for r in range(CT // 16):
                        buf[e0 + el, pl.ds(r * 16, 16)] = (
                            buf[e0 + el, pl.ds(r * 16, 16)] * splat)
                return 0
            lax.fori_loop(0, EK // 16, _scale_group, 0)
            pltpu.sync_copy(buf, acc.at[rows_v.at[i]], add=True)

        lax.fori_loop(0, EK, _zero_bufa, 0)
        for z in range(RPT // EK):
            pltpu.sync_copy(bufa, acc.at[pl.ds(sid * RPT + z * EK, EK)])
        plsc.subcore_barrier()

        def _blk(blk, _):
            pltpu.sync_copy(cols_h.at[sid, blk], cols_v)
            pltpu.sync_copy(rows_h.at[sid, blk], rows_v)
            pltpu.sync_copy(vals_h.at[sid, blk], vals_v)
            _gather_start(cols_v, 0, bufa, sema)

            def _pair(ii, _):
                i = ii * 2
                _gather_wait(cols_v, i, bufa, sema)
                _gather_start(cols_v, i + 1, bufb, semb)
                _scale_scatter(i, bufa)
                _gather_wait(cols_v, i + 1, bufb, semb)
                _gather_start(cols_v, i + 2, bufa, sema)
                _scale_scatter(i + 1, bufb)
                return 0
            lax.fori_loop(0, (CBLK - 1) // 2, _pair, 0)
            _gather_wait(cols_v, CBLK - 1, bufa, sema)
            _scale_scatter(CBLK - 1, bufa)
            return 0
        lax.fori_loop(0, NBLK, _blk, 0)
        plsc.subcore_barrier()

        @pl.when(sid < NS - 1)
        def _flush_main():
            pltpu.sync_copy(acc.at[pl.ds(sid * RPT, RPT)],
                            out_h.at[pl.ds(sid * RPT, RPT), pl.ds(cbase, CT)])

        @pl.when(sid == NS - 1)
        def _flush_last():
            nlast = NV - (NS - 1) * RPT
            pltpu.sync_copy(acc.at[pl.ds((NS - 1) * RPT, nlast)],
                            out_h.at[pl.ds((NS - 1) * RPT, nlast), pl.ds(cbase, CT)])
        plsc.subcore_barrier()
        return 0

    lax.fori_loop(0, TILES_PER_SC, _tile, 0)


def _spmm_sc(f3, rows, cols, vals):
    fn = pl.kernel(
        _spmm_body,
        mesh=plsc.VectorSubcoreMesh(core_axis_name="c", subcore_axis_name="s"),
        out_type=jax.ShapeDtypeStruct((NV, NCT * CT), jnp.float32),
        scratch_types=[
            pltpu.VMEM((CBLK, EK), jnp.int32),
            pltpu.VMEM((CBLK, EK), jnp.int32),
            pltpu.VMEM((CBLK, EK), jnp.float32),
            pltpu.VMEM((EK, CT), jnp.float32),
            pltpu.VMEM((EK, CT), jnp.float32),
            pltpu.VMEM_SHARED((RPAD, CT), jnp.float32),
            pltpu.SemaphoreType.DMA,
            pltpu.SemaphoreType.DMA,
        ],
    )
    return fn(f3, rows.reshape(NS, NBLK, CBLK, EK),
              cols.reshape(NS, NBLK, CBLK, EK),
              vals.reshape(NS, NBLK, CBLK, EK))


def _ka_body(hq, x1, gw, gb, o):
    for t in range(TOUT):
        g = lax.dot_general(gw[...], hq[0, t, :, :],
                            (((0,), (1,)), ((), ())),
                            preferred_element_type=jnp.float32)
        o[0, t, :, :] = jax.nn.relu(g + gb[...] + x1[0, :, t, :])


def _gcn_residual_relu(hq, x1, gw, gb):
    return pl.pallas_call(
        _ka_body,
        grid=(B, NVB),
        in_specs=[
            pl.BlockSpec((1, TOUT, VB, C), lambda b, v: (b, 0, v, 0)),
            pl.BlockSpec((1, C, TOUT, VB), lambda b, v: (b, 0, 0, v)),
            pl.BlockSpec((C, C), lambda b, v: (0, 0)),
            pl.BlockSpec((C, 1), lambda b, v: (0, 0)),
        ],
        out_specs=pl.BlockSpec((1, TOUT, C, VB), lambda b, v: (b, 0, 0, v)),
        out_shape=jax.ShapeDtypeStruct((B, TOUT, C, NV), jnp.float32),
    )(hq, x1, gw, gb)


def _kb_body(xr, w2, b2, gT, bT, o):
    for t in range(TOUT2):
        acc = b2[...]
        for k in range(KT):
            acc = acc + jnp.dot(w2[:, :, k], xr[0, t + k, :, :],
                                preferred_element_type=jnp.float32)
        x2 = jax.nn.relu(acc + xr[0, t + KT - 1, :, :])
        mean = jnp.mean(x2)
        var = jnp.mean((x2 - mean) ** 2)
        o[0, t, :, :] = (x2 - mean) * jax.lax.rsqrt(var + 1e-5) * gT[...] + bT[...]


def _conv2_ln(xr, w2, b2, gT, bT):
    return pl.pallas_call(
        _kb_body,
        grid=(B,),
        in_specs=[
            pl.BlockSpec((1, TOUT, C, NV), lambda b: (b, 0, 0, 0)),
            pl.BlockSpec((C, C, KT), lambda b: (0, 0, 0)),
            pl.BlockSpec((C, 1), lambda b: (0, 0)),
            pl.BlockSpec((C, NV), lambda b: (0, 0)),
            pl.BlockSpec((C, NV), lambda b: (0, 0)),
        ],
        out_specs=pl.BlockSpec((1, TOUT2, C, NV), lambda b: (b, 0, 0, 0)),
        out_shape=jax.ShapeDtypeStruct((B, TOUT2, C, NV), jnp.float32),
    )(xr, w2, b2, gT, bT)


def kernel(x, conv1_w, conv1_b, gcn_w, gcn_b, conv2_w, conv2_b,
           ln_gamma, ln_beta, filter_vals, filter_rows, filter_cols):
    x1 = _conv1_glu(x, conv1_w[:, :, :, 0], conv1_b.reshape(2 * C, 1))
    f3 = x1.reshape(NV, NCT * CT)
    h3 = _spmm_sc(f3, filter_rows, filter_cols, filter_vals)
    hq = h3.reshape(B, TOUT, NV, C)
    xr = _gcn_residual_relu(hq, x1, gcn_w, gcn_b.reshape(C, 1))
    out = _conv2_ln(xr, conv2_w[:, :, :, 0], conv2_b.reshape(C, 1),
                    ln_gamma.T, ln_beta.T)
    return out.transpose(0, 2, 1, 3)


# speedup vs baseline: 3.3484x; 1.1644x over previous
import functools

import jax
import jax.numpy as jnp
from jax import lax
from jax.experimental import pallas as pl
from jax.experimental.pallas import tpu as pltpu
from jax.experimental.pallas import tpu_sc as plsc

B, C, T, NV, KT, NNZ = 4, 32, 12, 10000, 3, 160000
TOUT = T - KT + 1
TOUT2 = TOUT - KT + 1


VB = 1280
NVB = -(-NV // VB)


def _k1_body(x, wp, wq, bp, bq, o):
    for t in range(TOUT):
        accp = bp[...]
        accq = bq[...]
        for k in range(KT):
            xk = x[0, :, t + k, :]
            accp = accp + jnp.dot(wp[k], xk, preferred_element_type=jnp.float32)
            accq = accq + jnp.dot(wq[k], xk, preferred_element_type=jnp.float32)
        o[0, :, t, :] = (accp + x[0, :, t + KT - 1, :]) * jax.nn.sigmoid(accq)


def _conv1_glu(x, wp, wq, bp, bq):
    wspec = pl.BlockSpec((KT, C, C), lambda b, v: (0, 0, 0))
    bspec = pl.BlockSpec((C, 1), lambda b, v: (0, 0))
    return pl.pallas_call(
        _k1_body,
        grid=(B, NVB),
        in_specs=[
            pl.BlockSpec((1, C, T, VB), lambda b, v: (b, 0, 0, v)),
            wspec, wspec, bspec, bspec,
        ],
        out_specs=pl.BlockSpec((1, C, TOUT, VB), lambda b, v: (b, 0, 0, v)),
        out_shape=jax.ShapeDtypeStruct((B, C, TOUT, NV), jnp.float32),
    )(x, wp, wq, bp, bq)


CT = 128
NCT = 1280 // CT
TILES_PER_SC = NCT // 2
NS = 16
EK = 80
EPT = NNZ // NS
NBLK = 5
CBLK = 25
RPAD = 10240
RPT = RPAD // NS


def _spmm_body(f3, rows_h, cols_h, vals_h, out_h,
               cols_v, rows_v, vals_v, bufa, bufb, acc, sema, semb):
    cid = lax.axis_index("c")
    sid = lax.axis_index("s")

    def _zero_bufa(i, _):
        for r in range(CT // 16):
            bufa[i, pl.ds(r * 16, 16)] = jnp.zeros((16,), jnp.float32)
        return 0

    def _tile(j, _unused):
        ct = cid * TILES_PER_SC + j
        cbase = pl.multiple_of(ct * CT, CT)

        def _gather_start(ref, i, buf, sem):
            pltpu.async_copy(f3.at[ref.at[i], pl.ds(cbase, CT)], buf, sem)

        def _gather_wait(ref, i, buf, sem):
            pltpu.make_async_copy(f3.at[ref.at[i], pl.ds(cbase, CT)],
                                  buf, sem).wait()

        dnums = lax.GatherDimensionNumbers(offset_dims=(),
                                           collapsed_slice_dims=(0,),
                                           start_index_map=(0,))

        def _scale_scatter(i, buf):
            def _scale_group(g, _):
                e0 = g * 16
                v16 = vals_v[i, pl.ds(e0, 16)]
                for el in range(16):
                    splat = lax.gather(
                        v16, jnp.full((16, 1), el, jnp.int32), dnums, (1,),
                        mode=lax.GatherScatterMode.PROMISE_IN_BOUNDS)
                    for r in range(CT // 16):
                        buf[e0 + el, pl.ds(r * 16, 16)] = (
                            buf[e0 + el, pl.ds(r * 16, 16)] * splat)
                return 0
            lax.fori_loop(0, EK // 16, _scale_group, 0)
            pltpu.sync_copy(buf, acc.at[rows_v.at[i]], add=True)

        lax.fori_loop(0, EK, _zero_bufa, 0)
        for z in range(RPT // EK):
            pltpu.sync_copy(bufa, acc.at[pl.ds(sid * RPT + z * EK, EK)])
        plsc.subcore_barrier()

        def _blk(blk, _):
            pltpu.sync_copy(cols_h.at[sid, blk], cols_v)
            pltpu.sync_copy(rows_h.at[sid, blk], rows_v)
            pltpu.sync_copy(vals_h.at[sid, blk], vals_v)
            _gather_start(cols_v, 0, bufa, sema)

            def _pair(ii, _):
                i = ii * 2
                _gather_wait(cols_v, i, bufa, sema)
                _gather_start(cols_v, i + 1, bufb, semb)
                _scale_scatter(i, bufa)
                _gather_wait(cols_v, i + 1, bufb, semb)
                _gather_start(cols_v, i + 2, bufa, sema)
                _scale_scatter(i + 1, bufb)
                return 0
            lax.fori_loop(0, (CBLK - 1) // 2, _pair, 0)
            _gather_wait(cols_v, CBLK - 1, bufa, sema)
            _scale_scatter(CBLK - 1, bufa)
            return 0
        lax.fori_loop(0, NBLK, _blk, 0)
        plsc.subcore_barrier()

        @pl.when(sid < NS - 1)
        def _flush_main():
            pltpu.sync_copy(acc.at[pl.ds(sid * RPT, RPT)],
                            out_h.at[pl.ds(sid * RPT, RPT), pl.ds(cbase, CT)])

        @pl.when(sid == NS - 1)
        def _flush_last():
            nlast = NV - (NS - 1) * RPT
            pltpu.sync_copy(acc.at[pl.ds((NS - 1) * RPT, nlast)],
                            out_h.at[pl.ds((NS - 1) * RPT, nlast), pl.ds(cbase, CT)])
        plsc.subcore_barrier()
        return 0

    lax.fori_loop(0, TILES_PER_SC, _tile, 0)


def _spmm_sc(f3, rows, cols, vals):
    fn = pl.kernel(
        _spmm_body,
        mesh=plsc.VectorSubcoreMesh(core_axis_name="c", subcore_axis_name="s"),
        out_type=jax.ShapeDtypeStruct((NV, NCT * CT), jnp.float32),
        scratch_types=[
            pltpu.VMEM((CBLK, EK), jnp.int32),
            pltpu.VMEM((CBLK, EK), jnp.int32),
            pltpu.VMEM((CBLK, EK), jnp.float32),
            pltpu.VMEM((EK, CT), jnp.float32),
            pltpu.VMEM((EK, CT), jnp.float32),
            pltpu.VMEM_SHARED((RPAD, CT), jnp.float32),
            pltpu.SemaphoreType.DMA,
            pltpu.SemaphoreType.DMA,
        ],
    )
    return fn(f3, rows.reshape(NS, NBLK, CBLK, EK),
              cols.reshape(NS, NBLK, CBLK, EK),
              vals.reshape(NS, NBLK, CBLK, EK))


def _ka_body(hq, x1, gw, gb, o):
    for t in range(TOUT):
        g = lax.dot_general(gw[...], hq[0, t, :, :],
                            (((0,), (1,)), ((), ())),
                            preferred_element_type=jnp.float32)
        o[0, t, :, :] = jax.nn.relu(g + gb[...] + x1[0, :, t, :])


def _gcn_residual_relu(hq, x1, gw, gb):
    return pl.pallas_call(
        _ka_body,
        grid=(B, NVB),
        in_specs=[
            pl.BlockSpec((1, TOUT, VB, C), lambda b, v: (b, 0, v, 0)),
            pl.BlockSpec((1, C, TOUT, VB), lambda b, v: (b, 0, 0, v)),
            pl.BlockSpec((C, C), lambda b, v: (0, 0)),
            pl.BlockSpec((C, 1), lambda b, v: (0, 0)),
        ],
        out_specs=pl.BlockSpec((1, TOUT, C, VB), lambda b, v: (b, 0, 0, v)),
        out_shape=jax.ShapeDtypeStruct((B, TOUT, C, NV), jnp.float32),
    )(hq, x1, gw, gb)


def _kb_body(xr, w2, b2, gT, bT, o):
    for t in range(TOUT2):
        acc = b2[...]
        for k in range(KT):
            acc = acc + jnp.dot(w2[k], xr[0, t + k, :, :],
                                preferred_element_type=jnp.float32)
        x2 = jax.nn.relu(acc + xr[0, t + KT - 1, :, :])
        mean = jnp.mean(x2)
        var = jnp.mean((x2 - mean) ** 2)
        o[0, t, :, :] = (x2 - mean) * jax.lax.rsqrt(var + 1e-5) * gT[...] + bT[...]


def _conv2_ln(xr, w2, b2, gT, bT):
    return pl.pallas_call(
        _kb_body,
        grid=(B,),
        in_specs=[
            pl.BlockSpec((1, TOUT, C, NV), lambda b: (b, 0, 0, 0)),
            pl.BlockSpec((KT, C, C), lambda b: (0, 0, 0)),
            pl.BlockSpec((C, 1), lambda b: (0, 0)),
            pl.BlockSpec((C, NV), lambda b: (0, 0)),
            pl.BlockSpec((C, NV), lambda b: (0, 0)),
        ],
        out_specs=pl.BlockSpec((1, TOUT2, C, NV), lambda b: (b, 0, 0, 0)),
        out_shape=jax.ShapeDtypeStruct((B, TOUT2, C, NV), jnp.float32),
    )(xr, w2, b2, gT, bT)


def kernel(x, conv1_w, conv1_b, gcn_w, gcn_b, conv2_w, conv2_b,
           ln_gamma, ln_beta, filter_vals, filter_rows, filter_cols):
    w1t = conv1_w[:, :, :, 0].transpose(2, 0, 1)
    x1 = _conv1_glu(x, w1t[:, :C, :], w1t[:, C:, :],
                    conv1_b[:C].reshape(C, 1), conv1_b[C:].reshape(C, 1))
    f3 = x1.reshape(NV, NCT * CT)
    h3 = _spmm_sc(f3, filter_rows, filter_cols, filter_vals)
    hq = h3.reshape(B, TOUT, NV, C)
    xr = _gcn_residual_relu(hq, x1, gcn_w, gcn_b.reshape(C, 1))
    out = _conv2_ln(xr, conv2_w[:, :, :, 0].transpose(2, 0, 1),
                    conv2_b.reshape(C, 1), ln_gamma.T, ln_beta.T)
    return out.transpose(0, 2, 1, 3)
